# baseline (speedup 1.0000x reference)
import functools

import jax
import jax.numpy as jnp
from jax import lax
from jax.experimental import pallas as pl
from jax.experimental.pallas import tpu as pltpu
from jax.experimental.pallas import tpu_sc as plsc

VOCAB = 30522
HIDDEN = 768
B, S, L = 64, 512, 100
VIS_DIM = 2048
EPS = 1e-12


_NC, _NS = 2, 16
_NW = _NC * _NS
_BS = B * S
_CHUNK = 64


def _sc_gather_body(nrows, table_hbm, idx_hbm, out_hbm, idx_v, buf_v,
                    gsem0, gsem1, ssem0, ssem1):
    per_w = nrows // _NW
    nchunk = per_w // _CHUNK
    wid = lax.axis_index("s") * _NC + lax.axis_index("c")
    base = wid * per_w
    pltpu.sync_copy(idx_hbm.at[pl.ds(base, per_w)], idx_v)

    gsems = (gsem0, gsem1)
    ssems = (ssem0, ssem1)

    def start_gather(i):
        bslot = i % 2
        return pltpu.async_copy(
            table_hbm.at[idx_v.at[pl.ds(i * _CHUNK, _CHUNK)]],
            buf_v.at[bslot], gsems[bslot])

    def start_scatter(i):
        bslot = i % 2
        return pltpu.async_copy(
            buf_v.at[bslot],
            out_hbm.at[pl.ds(base + i * _CHUNK, _CHUNK)], ssems[bslot])

    gathers = [None] * nchunk
    scatters = [None] * nchunk
    gathers[0] = start_gather(0)
    for i in range(nchunk):
        if i + 1 < nchunk:
            if i >= 1:
                scatters[i - 1].wait()
            gathers[i + 1] = start_gather(i + 1)
        gathers[i].wait()
        scatters[i] = start_scatter(i)
    scatters[nchunk - 2].wait()
    scatters[nchunk - 1].wait()


@functools.cache
def _sc_gather_kernel(nrows):
    return pl.kernel(
        functools.partial(_sc_gather_body, nrows),
        out_type=jax.ShapeDtypeStruct((nrows, HIDDEN), jnp.float32),
        mesh=plsc.VectorSubcoreMesh(core_axis_name="c", subcore_axis_name="s"),
        scratch_types=[
            pltpu.VMEM((nrows // _NW,), jnp.int32),
            pltpu.VMEM((2, _CHUNK, HIDDEN), jnp.float32),
            pltpu.SemaphoreType.DMA,
            pltpu.SemaphoreType.DMA,
            pltpu.SemaphoreType.DMA,
            pltpu.SemaphoreType.DMA,
        ],
    )


_CHV = 4
_NV = L // _CHV
_CHT = 32
_NT = S // _CHT


def _layer_norm3(x, gamma, beta):
    mu = jnp.mean(x, axis=-1, keepdims=True)
    xc = x - mu
    var = jnp.mean(xc * xc, axis=-1, keepdims=True)
    return xc * lax.rsqrt(var + EPS) * gamma + beta


def _tc_visual_body(vis_ref, vt_ref, tve_ref, w_ref, b_ref,
                    gamma_ref, beta_ref, out_ref):
    v = vis_ref[...].astype(jnp.bfloat16).reshape(_CHV * B, VIS_DIM)
    proj = jnp.dot(v, w_ref[...], preferred_element_type=jnp.float32)
    proj = proj.reshape(_CHV, B, HIDDEN)
    t = vt_ref[0].astype(jnp.float32)[:, :, None]
    tve0 = tve_ref[0][None, None, :]
    tve_d = (tve_ref[1] - tve_ref[0])[None, None, :]
    ve = proj + b_ref[...][None] + tve0 + t * tve_d
    out_ref[...] = _layer_norm3(ve, gamma_ref[...][None], beta_ref[...][None])


def _tc_visual(vis_t, vt3, tok_type_vis_emb, proj_Wb, proj_b2, gamma2, beta2):
    return pl.pallas_call(
        _tc_visual_body,
        grid=(_NV,),
        in_specs=[
            pl.BlockSpec((_CHV, B, VIS_DIM), lambda g: (g, 0, 0)),
            pl.BlockSpec((1, _CHV, B), lambda g: (g, 0, 0)),
            pl.BlockSpec((2, HIDDEN), lambda g: (0, 0)),
            pl.BlockSpec((VIS_DIM, HIDDEN), lambda g: (0, 0)),
            pl.BlockSpec((1, HIDDEN), lambda g: (0, 0)),
            pl.BlockSpec((1, HIDDEN), lambda g: (0, 0)),
            pl.BlockSpec((1, HIDDEN), lambda g: (0, 0)),
        ],
        out_specs=pl.BlockSpec((_CHV, B, HIDDEN),
                               lambda g: (S // _CHV + g, 0, 0)),
        out_shape=jax.ShapeDtypeStruct((S + L, B, HIDDEN), jnp.float32),
    )(vis_t, vt3, tok_type_vis_emb, proj_Wb, proj_b2, gamma2, beta2)


def _tc_text_body(buf_ref, gw_ref, pos_ref, tt_ref, tte_ref,
                  gamma_ref, beta_ref, out_ref):
    del buf_ref
    t = tt_ref[0].astype(jnp.float32)[:, :, None]
    tte0 = tte_ref[0][None, None, :]
    tte_d = (tte_ref[1] - tte_ref[0])[None, None, :]
    e = gw_ref[...] + pos_ref[0][:, None, :] + tte0 + t * tte_d
    out_ref[...] = _layer_norm3(e, gamma_ref[...][None], beta_ref[...][None])


_NHALF = 2
_SH = S // _NHALF
_NTH = _SH // _CHT


def _tc_text(buf, gw_h, pos3, tt3, tok_type_emb, gamma2, beta2, off):
    return pl.pallas_call(
        _tc_text_body,
        grid=(_NTH,),
        in_specs=[
            pl.BlockSpec(memory_space=pl.ANY),
            pl.BlockSpec((_CHT, B, HIDDEN), lambda g: (g, 0, 0)),
            pl.BlockSpec((1, _CHT, HIDDEN), lambda g: (g + off, 0, 0)),
            pl.BlockSpec((1, _CHT, B), lambda g: (g + off, 0, 0)),
            pl.BlockSpec((2, HIDDEN), lambda g: (0, 0)),
            pl.BlockSpec((1, HIDDEN), lambda g: (0, 0)),
            pl.BlockSpec((1, HIDDEN), lambda g: (0, 0)),
        ],
        out_specs=pl.BlockSpec((_CHT, B, HIDDEN), lambda g: (g + off, 0, 0)),
        out_shape=jax.ShapeDtypeStruct((S + L, B, HIDDEN), jnp.float32),
        input_output_aliases={0: 0},
    )(buf, gw_h, pos3, tt3, tok_type_emb, gamma2, beta2)


def kernel(input_ids, token_type_ids, visual_embeddings, visual_embeddings_type,
           word_emb, pos_emb, tok_type_emb, tok_type_vis_emb,
           proj_W, proj_b, ln_gamma, ln_beta):
    idx_t = input_ids.astype(jnp.int32).T.reshape(_NHALF, _SH * B)
    tt3 = token_type_ids.astype(jnp.int32).T.reshape(_NT, _CHT, B)
    vt3 = visual_embeddings_type.astype(jnp.int32).T.reshape(_NV, _CHV, B)
    vis_t = jnp.transpose(visual_embeddings, (1, 0, 2))
    pos3 = pos_emb.reshape(_NT, _CHT, HIDDEN)
    gamma2 = ln_gamma.reshape(1, HIDDEN)
    beta2 = ln_beta.reshape(1, HIDDEN)
    gw = [_sc_gather_kernel(_SH * B)(word_emb, idx_t[h]).reshape(_SH, B, HIDDEN)
          for h in range(_NHALF)]
    buf = _tc_visual(vis_t, vt3, tok_type_vis_emb,
                     proj_W.astype(jnp.bfloat16), proj_b.reshape(1, HIDDEN),
                     gamma2, beta2)
    for h in range(_NHALF):
        buf = _tc_text(buf, gw[h], pos3, tt3, tok_type_emb, gamma2, beta2,
                       h * _NTH)
    return jnp.transpose(buf, (1, 0, 2))
